# baseline (speedup 1.0000x reference)
import functools

import jax
import jax.numpy as jnp
import numpy as np
from jax import lax
from jax.experimental import pallas as pl
from jax.experimental.pallas import tpu as pltpu
from jax.experimental.pallas import tpu_sc as plsc

CATS = 81
SCALES = 5
DIM = 256
T = 0.07
N_IDX = 100000
N_PER_LVL = 20000
CHUNK = 128
N_WORKERS = 32
N_CHUNKS = 25
PER_W = N_CHUNKS * CHUNK
LROWS = 168
ROWS = 512

_pos = np.arange(N_WORKERS * PER_W)
_w = _pos // PER_W
_l0 = (_w * PER_W) // N_PER_LVL
_lvl = _pos // N_PER_LVL
_SEGBASE_LOCAL = np.where(
    _pos < N_IDX, (_lvl - _l0) * CATS, 2 * CATS).astype(np.int32)
_L0 = [int((w * PER_W) // N_PER_LVL) for w in range(N_WORKERS)]


def _sc_segment_sums(feats, packed, targets):
    mesh = plsc.VectorSubcoreMesh(core_axis_name="c", subcore_axis_name="s")
    NC = 2

    @functools.partial(
        pl.kernel,
        mesh=mesh,
        out_type=[
            jax.ShapeDtypeStruct((N_WORKERS, LROWS, DIM), jnp.float32),
            jax.ShapeDtypeStruct((N_WORKERS, LROWS, 16), jnp.float32),
        ],
        scratch_types=[
            pltpu.VMEM((N_CHUNKS, CHUNK), jnp.int32),
            pltpu.VMEM((CHUNK,), jnp.int32),
            pltpu.VMEM((CHUNK,), jnp.int32),
            pltpu.VMEM((CHUNK, DIM), jnp.float32),
            pltpu.VMEM((CHUNK,), jnp.int32),
            pltpu.VMEM((LROWS, DIM), jnp.float32),
            pltpu.VMEM((LROWS, 16), jnp.float32),
            pltpu.VMEM((16, 16), jnp.int32),
        ],
        compiler_params=pltpu.CompilerParams(
            needs_layout_passes=False, disable_bounds_checks=True),
    )
    def k(feats_hbm, pk_hbm, tgt_hbm, out_sums, out_cnt,
          pk_v, idxc_v, seg_v, rows_v, lab_v, acc_v, cnt_v, rot_v):
        cid = lax.axis_index("c")
        sid = lax.axis_index("s")
        wid = sid * NC + cid

        zeros16 = jnp.zeros((16,), jnp.float32)

        def _fill_acc(r, _):
            for m in range(DIM // 16):
                acc_v[r, pl.ds(m * 16, 16)] = zeros16
            cnt_v[r, pl.ds(0, 16)] = zeros16
            return 0
        lax.fori_loop(0, LROWS, _fill_acc, 0)

        iota16 = lax.iota(jnp.int32, 16)
        ones16 = jnp.ones((16,), jnp.float32)
        for r in range(16):
            rot_v[r, pl.ds(0, 16)] = jnp.bitwise_and(iota16 + r, 15)

        pltpu.sync_copy(pk_hbm.at[wid], pk_v)

        def chunk_body(j, _):
            for m in range(CHUNK // 16):
                sl = pl.ds(m * 16, 16)
                idxc_v[sl] = jnp.bitwise_and(pk_v[j, sl], (1 << 20) - 1)
            pltpu.sync_copy(tgt_hbm.at[idxc_v], lab_v)
            pltpu.sync_copy(feats_hbm.at[idxc_v], rows_v)
            for m in range(CHUNK // 16):
                sl = pl.ds(m * 16, 16)
                seg_v[sl] = lax.shift_right_logical(pk_v[j, sl], 20) + lab_v[sl]
            nseg = []
            nrid = []
            nmsk = []
            for g in range(CHUNK // 16):
                slg = pl.ds(g * 16, 16)
                seg16 = seg_v[slg]
                rid16 = iota16 + (g * 16)
                nseg.append(seg16)
                nrid.append(rid16)
                msk16 = seg16 < (2 * CATS)
                nmsk.append(msk16)
                plsc.addupdate_scatter(cnt_v, [seg16, iota16], ones16, mask=msk16)

            def rbody(r, _):
                rot = rot_v[r, pl.ds(0, 16)]
                for k in range(0, DIM // 16, 2):
                    dsp_a = rot + (k * 16)
                    dsp_b = rot + ((k + 1) * 16)
                    vals_a = [plsc.load_gather(rows_v, [nrid[g], dsp_a])
                              for g in range(CHUNK // 16)]
                    vals_b = [plsc.load_gather(rows_v, [nrid[g], dsp_b])
                              for g in range(CHUNK // 16)]
                    for g in range(CHUNK // 16):
                        plsc.addupdate_scatter(acc_v, [nseg[g], dsp_a], vals_a[g], mask=nmsk[g])
                    for g in range(CHUNK // 16):
                        plsc.addupdate_scatter(acc_v, [nseg[g], dsp_b], vals_b[g], mask=nmsk[g])
                return 0
            lax.fori_loop(0, 16, rbody, 0)
            return 0

        lax.fori_loop(0, N_CHUNKS, chunk_body, 0)

        pltpu.sync_copy(acc_v, out_sums.at[wid])
        pltpu.sync_copy(cnt_v, out_cnt.at[wid])

    return k(feats, packed, targets)


def _tc_loss_body(psum_ref, pcnt_ref, proto_ref, out_ref):
    blocks = []
    cblocks = []
    for l in range(SCALES):
        b = jnp.zeros((CATS, DIM), jnp.float32)
        c = jnp.zeros((CATS, 16), jnp.float32)
        for w in range(N_WORKERS):
            if _L0[w] == l:
                b = b + psum_ref[w, 0:CATS, :]
                c = c + pcnt_ref[w, 0:CATS, :]
            elif _L0[w] == l - 1:
                b = b + psum_ref[w, CATS:2 * CATS, :]
                c = c + pcnt_ref[w, CATS:2 * CATS, :]
        blocks.append(b)
        cblocks.append(c)
    pad = ROWS - CATS * SCALES
    sums = jnp.concatenate(blocks + [jnp.zeros((pad, DIM), jnp.float32)], axis=0)
    cntf = jnp.concatenate(cblocks + [jnp.zeros((pad, 16), jnp.float32)], axis=0)
    cnt = jnp.sum(cntf, axis=1, keepdims=True)

    occ = (cnt > 0.0).astype(jnp.float32)
    means = sums / jnp.maximum(cnt, 1.0)
    delta = jnp.where(cnt > 0.0, means, jnp.float32(0.01))

    def _norm(x):
        n2 = jnp.sum(x * x, axis=1, keepdims=True)
        return x * lax.rsqrt(jnp.maximum(n2, jnp.float32(1e-30)))

    v1 = _norm(proto_ref[...])
    v2 = _norm(delta)
    logits = lax.dot_general(v1, v2, (((1,), (1,)), ((), ())),
                             preferred_element_type=jnp.float32) / T

    r = lax.broadcasted_iota(jnp.int32, (ROWS, ROWS), 0)
    q = lax.broadcasted_iota(jnp.int32, (ROWS, ROWS), 1)
    s_of_r = jnp.mod(r, SCALES)
    in_block = (q // CATS) == s_of_r
    ml = jnp.where(in_block, logits, jnp.float32(-1e30))
    mx = jnp.max(ml, axis=1, keepdims=True)
    lse = jnp.log(jnp.sum(jnp.exp(ml - mx), axis=1, keepdims=True)) + mx

    tcol = s_of_r * CATS + jnp.mod(r, CATS)
    tval = jnp.sum(jnp.where(q == tcol, logits, 0.0), axis=1, keepdims=True)
    ce = lse - tval

    perm = s_of_r * CATS + r // SCALES
    occ_row = jnp.reshape(occ, (1, ROWS))
    mrow = jnp.sum(jnp.where(q == perm, occ_row, 0.0), axis=1, keepdims=True)
    rr = lax.broadcasted_iota(jnp.int32, (ROWS, 1), 0)
    mrow = jnp.where(rr < CATS * SCALES, mrow, 0.0)

    num = jnp.sum(ce * mrow, axis=0, keepdims=True)
    den = jnp.maximum(jnp.sum(mrow, axis=0, keepdims=True), 1.0)
    out_ref[...] = num / den


def _tc_loss(psums, pcnts, proto_pad):
    return pl.pallas_call(
        _tc_loss_body,
        out_shape=jax.ShapeDtypeStruct((1, 1), jnp.float32),
    )(psums, pcnts, proto_pad)


def kernel(cls_feats, cls_targets, lvl_idx, prototypes):
    padded = N_WORKERS * PER_W

    flat = lvl_idx.reshape(-1).astype(jnp.int32)
    idx_pad = jnp.concatenate(
        [flat, jnp.zeros((padded - N_IDX,), jnp.int32)]
    ).reshape(N_WORKERS, N_CHUNKS, CHUNK)
    segbase = jnp.asarray(_SEGBASE_LOCAL).reshape(N_WORKERS, N_CHUNKS, CHUNK)
    targets = cls_targets.astype(jnp.int32)

    packed = jnp.bitwise_or(jnp.left_shift(segbase, 20), idx_pad)
    psums, pcnts = _sc_segment_sums(cls_feats, packed, targets)

    proto = prototypes.reshape(CATS * SCALES, DIM)
    proto_pad = jnp.concatenate(
        [proto, jnp.zeros((ROWS - CATS * SCALES, DIM), jnp.float32)], axis=0)

    loss = _tc_loss(psums, pcnts, proto_pad)
    return loss.reshape(())


# speedup vs baseline: 1.0812x; 1.0812x over previous
import functools

import jax
import jax.numpy as jnp
import numpy as np
from jax import lax
from jax.experimental import pallas as pl
from jax.experimental.pallas import tpu as pltpu
from jax.experimental.pallas import tpu_sc as plsc

CATS = 81
SCALES = 5
DIM = 256
T = 0.07
N_IDX = 100000
N_PER_LVL = 20000
CHUNK = 128
N_WORKERS = 32
N_CHUNKS = 25
PER_W = N_CHUNKS * CHUNK
LROWS = 168
ROWS = 512

_pos = np.arange(N_WORKERS * PER_W)
_w = _pos // PER_W
_l0 = (_w * PER_W) // N_PER_LVL
_lvl = _pos // N_PER_LVL
_SEGBASE_LOCAL = np.where(
    _pos < N_IDX, (_lvl - _l0) * CATS, 2 * CATS).astype(np.int32)
_L0 = [int((w * PER_W) // N_PER_LVL) for w in range(N_WORKERS)]


def _sc_segment_sums(feats, packed, targets):
    mesh = plsc.VectorSubcoreMesh(core_axis_name="c", subcore_axis_name="s")
    NC = 2

    @functools.partial(
        pl.kernel,
        mesh=mesh,
        out_type=[
            jax.ShapeDtypeStruct((N_WORKERS, LROWS, DIM), jnp.float32),
            jax.ShapeDtypeStruct((N_WORKERS, LROWS, 16), jnp.float32),
        ],
        scratch_types=[
            pltpu.VMEM((N_CHUNKS, CHUNK), jnp.int32),
            pltpu.VMEM((CHUNK,), jnp.int32),
            pltpu.VMEM((CHUNK,), jnp.int32),
            pltpu.VMEM((CHUNK, DIM), jnp.float32),
            pltpu.VMEM((CHUNK,), jnp.int32),
            pltpu.VMEM((LROWS, DIM), jnp.float32),
            pltpu.VMEM((LROWS, 16), jnp.float32),
            pltpu.VMEM((16, 16), jnp.int32),
            pltpu.SemaphoreType.DMA,
        ],
        compiler_params=pltpu.CompilerParams(
            needs_layout_passes=False, disable_bounds_checks=True),
    )
    def k(feats_hbm, pk_hbm, tgt_hbm, out_sums, out_cnt,
          pk_v, idxc_v, seg_v, rows_v, lab_v, acc_v, cnt_v, rot_v, sem_r):
        cid = lax.axis_index("c")
        sid = lax.axis_index("s")
        wid = sid * NC + cid

        zeros16 = jnp.zeros((16,), jnp.float32)

        def _fill_acc(r, _):
            for m in range(DIM // 16):
                acc_v[r, pl.ds(m * 16, 16)] = zeros16
            cnt_v[r, pl.ds(0, 16)] = zeros16
            return 0
        lax.fori_loop(0, LROWS, _fill_acc, 0)

        iota16 = lax.iota(jnp.int32, 16)
        ones16 = jnp.ones((16,), jnp.float32)
        for r in range(16):
            rot_v[r, pl.ds(0, 16)] = jnp.bitwise_and(iota16 + r, 15)

        pltpu.sync_copy(pk_hbm.at[wid], pk_v)

        def chunk_body(j, _):
            for m in range(CHUNK // 16):
                sl = pl.ds(m * 16, 16)
                idxc_v[sl] = jnp.bitwise_and(pk_v[j, sl], (1 << 20) - 1)
            cpr = pltpu.async_copy(feats_hbm.at[idxc_v], rows_v, sem_r)
            pltpu.sync_copy(tgt_hbm.at[idxc_v], lab_v)
            for m in range(CHUNK // 16):
                sl = pl.ds(m * 16, 16)
                seg_v[sl] = lax.shift_right_logical(pk_v[j, sl], 20) + lab_v[sl]
            cpr.wait()
            nseg = []
            nrid = []
            nmsk = []
            for g in range(CHUNK // 16):
                slg = pl.ds(g * 16, 16)
                seg16 = seg_v[slg]
                rid16 = iota16 + (g * 16)
                nseg.append(seg16)
                nrid.append(rid16)
                msk16 = seg16 < (2 * CATS)
                nmsk.append(msk16)
                plsc.addupdate_scatter(cnt_v, [seg16, iota16], ones16, mask=msk16)

            def rbody(r, _):
                rot = rot_v[r, pl.ds(0, 16)]
                for k in range(0, DIM // 16, 2):
                    dsp_a = rot + (k * 16)
                    dsp_b = rot + ((k + 1) * 16)
                    vals_a = [plsc.load_gather(rows_v, [nrid[g], dsp_a])
                              for g in range(CHUNK // 16)]
                    vals_b = [plsc.load_gather(rows_v, [nrid[g], dsp_b])
                              for g in range(CHUNK // 16)]
                    for g in range(CHUNK // 16):
                        plsc.addupdate_scatter(acc_v, [nseg[g], dsp_a], vals_a[g], mask=nmsk[g])
                    for g in range(CHUNK // 16):
                        plsc.addupdate_scatter(acc_v, [nseg[g], dsp_b], vals_b[g], mask=nmsk[g])
                return 0
            lax.fori_loop(0, 16, rbody, 0)
            return 0

        lax.fori_loop(0, N_CHUNKS, chunk_body, 0)

        pltpu.sync_copy(acc_v, out_sums.at[wid])
        pltpu.sync_copy(cnt_v, out_cnt.at[wid])

    return k(feats, packed, targets)


def _tc_loss_body(psum_ref, pcnt_ref, proto_ref, out_ref):
    blocks = []
    cblocks = []
    for l in range(SCALES):
        b = jnp.zeros((CATS, DIM), jnp.float32)
        c = jnp.zeros((CATS, 16), jnp.float32)
        for w in range(N_WORKERS):
            if _L0[w] == l:
                b = b + psum_ref[w, 0:CATS, :]
                c = c + pcnt_ref[w, 0:CATS, :]
            elif _L0[w] == l - 1:
                b = b + psum_ref[w, CATS:2 * CATS, :]
                c = c + pcnt_ref[w, CATS:2 * CATS, :]
        blocks.append(b)
        cblocks.append(c)
    pad = ROWS - CATS * SCALES
    sums = jnp.concatenate(blocks + [jnp.zeros((pad, DIM), jnp.float32)], axis=0)
    cntf = jnp.concatenate(cblocks + [jnp.zeros((pad, 16), jnp.float32)], axis=0)
    cnt = jnp.sum(cntf, axis=1, keepdims=True)

    occ = (cnt > 0.0).astype(jnp.float32)
    means = sums / jnp.maximum(cnt, 1.0)
    delta = jnp.where(cnt > 0.0, means, jnp.float32(0.01))

    def _norm(x):
        n2 = jnp.sum(x * x, axis=1, keepdims=True)
        return x * lax.rsqrt(jnp.maximum(n2, jnp.float32(1e-30)))

    v1 = _norm(proto_ref[...])
    v2 = _norm(delta)
    logits = lax.dot_general(v1, v2, (((1,), (1,)), ((), ())),
                             preferred_element_type=jnp.float32) / T

    r = lax.broadcasted_iota(jnp.int32, (ROWS, ROWS), 0)
    q = lax.broadcasted_iota(jnp.int32, (ROWS, ROWS), 1)
    s_of_r = jnp.mod(r, SCALES)
    in_block = (q // CATS) == s_of_r
    ml = jnp.where(in_block, logits, jnp.float32(-1e30))
    mx = jnp.max(ml, axis=1, keepdims=True)
    lse = jnp.log(jnp.sum(jnp.exp(ml - mx), axis=1, keepdims=True)) + mx

    tcol = s_of_r * CATS + jnp.mod(r, CATS)
    tval = jnp.sum(jnp.where(q == tcol, logits, 0.0), axis=1, keepdims=True)
    ce = lse - tval

    perm = s_of_r * CATS + r // SCALES
    occ_row = jnp.reshape(occ, (1, ROWS))
    mrow = jnp.sum(jnp.where(q == perm, occ_row, 0.0), axis=1, keepdims=True)
    rr = lax.broadcasted_iota(jnp.int32, (ROWS, 1), 0)
    mrow = jnp.where(rr < CATS * SCALES, mrow, 0.0)

    num = jnp.sum(ce * mrow, axis=0, keepdims=True)
    den = jnp.maximum(jnp.sum(mrow, axis=0, keepdims=True), 1.0)
    out_ref[...] = num / den


def _tc_loss(psums, pcnts, proto_pad):
    return pl.pallas_call(
        _tc_loss_body,
        out_shape=jax.ShapeDtypeStruct((1, 1), jnp.float32),
    )(psums, pcnts, proto_pad)


def kernel(cls_feats, cls_targets, lvl_idx, prototypes):
    padded = N_WORKERS * PER_W

    flat = lvl_idx.reshape(-1).astype(jnp.int32)
    idx_pad = jnp.concatenate(
        [flat, jnp.zeros((padded - N_IDX,), jnp.int32)]
    ).reshape(N_WORKERS, N_CHUNKS, CHUNK)
    segbase = jnp.asarray(_SEGBASE_LOCAL).reshape(N_WORKERS, N_CHUNKS, CHUNK)
    targets = cls_targets.astype(jnp.int32)

    packed = jnp.bitwise_or(jnp.left_shift(segbase, 20), idx_pad)
    psums, pcnts = _sc_segment_sums(cls_feats, packed, targets)

    proto = prototypes.reshape(CATS * SCALES, DIM)
    proto_pad = jnp.concatenate(
        [proto, jnp.zeros((ROWS - CATS * SCALES, DIM), jnp.float32)], axis=0)

    loss = _tc_loss(psums, pcnts, proto_pad)
    return loss.reshape(())


# speedup vs baseline: 1.2060x; 1.1154x over previous
import functools

import jax
import jax.numpy as jnp
import numpy as np
from jax import lax
from jax.experimental import pallas as pl
from jax.experimental.pallas import tpu as pltpu
from jax.experimental.pallas import tpu_sc as plsc

CATS = 81
SCALES = 5
DIM = 256
T = 0.07
N_IDX = 100000
N_PER_LVL = 20000
CHUNK = 64
N_WORKERS = 32
N_CHUNKS = 50
PER_W = N_CHUNKS * CHUNK
LROWS = 168
ROWS = 512

_pos = np.arange(N_WORKERS * PER_W)
_w = _pos // PER_W
_l0 = (_w * PER_W) // N_PER_LVL
_lvl = _pos // N_PER_LVL
_SEGBASE_LOCAL = np.where(
    _pos < N_IDX, (_lvl - _l0) * CATS, 2 * CATS).astype(np.int32)
_L0 = [int((w * PER_W) // N_PER_LVL) for w in range(N_WORKERS)]


def _sc_segment_sums(feats, packed, targets):
    mesh = plsc.VectorSubcoreMesh(core_axis_name="c", subcore_axis_name="s")
    NC = 2

    @functools.partial(
        pl.kernel,
        mesh=mesh,
        out_type=[
            jax.ShapeDtypeStruct((N_WORKERS, LROWS, DIM), jnp.float32),
            jax.ShapeDtypeStruct((N_WORKERS, LROWS, 16), jnp.float32),
        ],
        scratch_types=[
            pltpu.VMEM((N_CHUNKS, CHUNK), jnp.int32),
            pltpu.VMEM((CHUNK,), jnp.int32),
            pltpu.VMEM((CHUNK,), jnp.int32),
            pltpu.VMEM((CHUNK,), jnp.int32),
            pltpu.VMEM((CHUNK,), jnp.int32),
            pltpu.VMEM((CHUNK, DIM), jnp.float32),
            pltpu.VMEM((CHUNK, DIM), jnp.float32),
            pltpu.VMEM((CHUNK,), jnp.int32),
            pltpu.VMEM((LROWS, DIM), jnp.float32),
            pltpu.VMEM((LROWS, 16), jnp.float32),
            pltpu.VMEM((16, 16), jnp.int32),
            pltpu.SemaphoreType.DMA,
            pltpu.SemaphoreType.DMA,
        ],
        compiler_params=pltpu.CompilerParams(
            needs_layout_passes=False, disable_bounds_checks=True),
    )
    def k(feats_hbm, pk_hbm, tgt_hbm, out_sums, out_cnt,
          pk_v, idxc0_v, idxc1_v, seg0_v, seg1_v, rows0_v, rows1_v,
          lab_v, acc_v, cnt_v, rot_v, sem_r0, sem_r1):
        idxc_b = (idxc0_v, idxc1_v)
        seg_b = (seg0_v, seg1_v)
        rows_bb = (rows0_v, rows1_v)
        sem_b = (sem_r0, sem_r1)
        cid = lax.axis_index("c")
        sid = lax.axis_index("s")
        wid = sid * NC + cid

        zeros16 = jnp.zeros((16,), jnp.float32)

        def _fill_acc(r, _):
            for m in range(DIM // 16):
                acc_v[r, pl.ds(m * 16, 16)] = zeros16
            cnt_v[r, pl.ds(0, 16)] = zeros16
            return 0
        lax.fori_loop(0, LROWS, _fill_acc, 0)

        iota16 = lax.iota(jnp.int32, 16)
        ones16 = jnp.ones((16,), jnp.float32)
        for r in range(16):
            rot_v[r, pl.ds(0, 16)] = jnp.bitwise_and(iota16 + r, 15)

        pltpu.sync_copy(pk_hbm.at[wid], pk_v)

        def issue(j, b):
            idxc_v = idxc_b[b]
            seg_v = seg_b[b]
            for m in range(CHUNK // 16):
                sl = pl.ds(m * 16, 16)
                idxc_v[sl] = jnp.bitwise_and(pk_v[j, sl], (1 << 20) - 1)
            cpr = pltpu.async_copy(feats_hbm.at[idxc_v], rows_bb[b], sem_b[b])
            pltpu.sync_copy(tgt_hbm.at[idxc_v], lab_v)
            for m in range(CHUNK // 16):
                sl = pl.ds(m * 16, 16)
                seg_v[sl] = lax.shift_right_logical(pk_v[j, sl], 20) + lab_v[sl]
            return cpr

        def accumulate(b, cpr):
            seg_v = seg_b[b]
            rows_v = rows_bb[b]
            cpr.wait()
            nseg = []
            nrid = []
            nmsk = []
            for g in range(CHUNK // 16):
                slg = pl.ds(g * 16, 16)
                seg16 = seg_v[slg]
                rid16 = iota16 + (g * 16)
                nseg.append(seg16)
                nrid.append(rid16)
                msk16 = seg16 < (2 * CATS)
                nmsk.append(msk16)
                plsc.addupdate_scatter(cnt_v, [seg16, iota16], ones16, mask=msk16)

            def rbody(r, _):
                rot = rot_v[r, pl.ds(0, 16)]
                for k in range(0, DIM // 16, 2):
                    dsp_a = rot + (k * 16)
                    dsp_b = rot + ((k + 1) * 16)
                    vals_a = [plsc.load_gather(rows_v, [nrid[g], dsp_a])
                              for g in range(CHUNK // 16)]
                    vals_b = [plsc.load_gather(rows_v, [nrid[g], dsp_b])
                              for g in range(CHUNK // 16)]
                    for g in range(CHUNK // 16):
                        plsc.addupdate_scatter(acc_v, [nseg[g], dsp_a], vals_a[g], mask=nmsk[g])
                    for g in range(CHUNK // 16):
                        plsc.addupdate_scatter(acc_v, [nseg[g], dsp_b], vals_b[g], mask=nmsk[g])
                return 0
            lax.fori_loop(0, 16, rbody, 0)

        def pair_body(t, _):
            c0 = t * 2
            cp0 = issue(c0, 0)
            cp1 = issue(c0 + 1, 1)
            accumulate(0, cp0)
            accumulate(1, cp1)
            return 0

        lax.fori_loop(0, N_CHUNKS // 2, pair_body, 0)
        if N_CHUNKS % 2:
            cp_last = issue(N_CHUNKS - 1, 0)
            accumulate(0, cp_last)

        pltpu.sync_copy(acc_v, out_sums.at[wid])
        pltpu.sync_copy(cnt_v, out_cnt.at[wid])

    return k(feats, packed, targets)


def _tc_loss_body(psum_ref, pcnt_ref, proto_ref, out_ref):
    blocks = []
    cblocks = []
    for l in range(SCALES):
        b = jnp.zeros((CATS, DIM), jnp.float32)
        c = jnp.zeros((CATS, 16), jnp.float32)
        for w in range(N_WORKERS):
            if _L0[w] == l:
                b = b + psum_ref[w, 0:CATS, :]
                c = c + pcnt_ref[w, 0:CATS, :]
            elif _L0[w] == l - 1:
                b = b + psum_ref[w, CATS:2 * CATS, :]
                c = c + pcnt_ref[w, CATS:2 * CATS, :]
        blocks.append(b)
        cblocks.append(c)
    pad = ROWS - CATS * SCALES
    sums = jnp.concatenate(blocks + [jnp.zeros((pad, DIM), jnp.float32)], axis=0)
    cntf = jnp.concatenate(cblocks + [jnp.zeros((pad, 16), jnp.float32)], axis=0)
    cnt = jnp.sum(cntf, axis=1, keepdims=True)

    occ = (cnt > 0.0).astype(jnp.float32)
    means = sums / jnp.maximum(cnt, 1.0)
    delta = jnp.where(cnt > 0.0, means, jnp.float32(0.01))

    def _norm(x):
        n2 = jnp.sum(x * x, axis=1, keepdims=True)
        return x * lax.rsqrt(jnp.maximum(n2, jnp.float32(1e-30)))

    v1 = _norm(proto_ref[...])
    v2 = _norm(delta)
    logits = lax.dot_general(v1, v2, (((1,), (1,)), ((), ())),
                             preferred_element_type=jnp.float32) / T

    r = lax.broadcasted_iota(jnp.int32, (ROWS, ROWS), 0)
    q = lax.broadcasted_iota(jnp.int32, (ROWS, ROWS), 1)
    s_of_r = jnp.mod(r, SCALES)
    in_block = (q // CATS) == s_of_r
    ml = jnp.where(in_block, logits, jnp.float32(-1e30))
    mx = jnp.max(ml, axis=1, keepdims=True)
    lse = jnp.log(jnp.sum(jnp.exp(ml - mx), axis=1, keepdims=True)) + mx

    tcol = s_of_r * CATS + jnp.mod(r, CATS)
    tval = jnp.sum(jnp.where(q == tcol, logits, 0.0), axis=1, keepdims=True)
    ce = lse - tval

    perm = s_of_r * CATS + r // SCALES
    occ_row = jnp.reshape(occ, (1, ROWS))
    mrow = jnp.sum(jnp.where(q == perm, occ_row, 0.0), axis=1, keepdims=True)
    rr = lax.broadcasted_iota(jnp.int32, (ROWS, 1), 0)
    mrow = jnp.where(rr < CATS * SCALES, mrow, 0.0)

    num = jnp.sum(ce * mrow, axis=0, keepdims=True)
    den = jnp.maximum(jnp.sum(mrow, axis=0, keepdims=True), 1.0)
    out_ref[...] = num / den


def _tc_loss(psums, pcnts, proto_pad):
    return pl.pallas_call(
        _tc_loss_body,
        out_shape=jax.ShapeDtypeStruct((1, 1), jnp.float32),
    )(psums, pcnts, proto_pad)


def kernel(cls_feats, cls_targets, lvl_idx, prototypes):
    padded = N_WORKERS * PER_W

    flat = lvl_idx.reshape(-1).astype(jnp.int32)
    idx_pad = jnp.concatenate(
        [flat, jnp.zeros((padded - N_IDX,), jnp.int32)]
    ).reshape(N_WORKERS, N_CHUNKS, CHUNK)
    segbase = jnp.asarray(_SEGBASE_LOCAL).reshape(N_WORKERS, N_CHUNKS, CHUNK)
    targets = cls_targets.astype(jnp.int32)

    packed = jnp.bitwise_or(jnp.left_shift(segbase, 20), idx_pad)
    psums, pcnts = _sc_segment_sums(cls_feats, packed, targets)

    proto = prototypes.reshape(CATS * SCALES, DIM)
    proto_pad = jnp.concatenate(
        [proto, jnp.zeros((ROWS - CATS * SCALES, DIM), jnp.float32)], axis=0)

    loss = _tc_loss(psums, pcnts, proto_pad)
    return loss.reshape(())
